# baseline (speedup 1.0000x reference)
import jax
import jax.numpy as jnp
from jax import lax
from jax.experimental import pallas as pl
from jax.experimental.pallas import tpu as pltpu
from jax.experimental.pallas import tpu_sc as plsc

N_NODES = 10000
D = 128
F = 64
NC, NS = 2, 16
CH = 128
N_PAD = 10240
RPT = N_PAD // NS
G = 16
BLK = 1024


def _deg_body(dstp, out, ones_v, zb, didx, acc, sdm):
    core = lax.axis_index("c")
    sub = lax.axis_index("s")
    wid = core * NS + sub
    nch = dstp.shape[0]
    cpw = nch // (NC * NS)

    @pl.loop(0, CH)
    def _fill(i):
        ones_v[i, :] = jnp.ones((16,), jnp.float32)
        zb[i, :] = jnp.zeros((16,), jnp.float32)

    @pl.loop(0, RPT // CH)
    def _z(k):
        pltpu.sync_copy(zb, acc.at[pl.ds(sub * RPT + k * CH, CH)])

    pltpu.sync_copy(dstp.at[pl.ds(wid * cpw, cpw)], didx)
    plsc.subcore_barrier()

    @pl.loop(0, cpw)
    def _go(j):
        pltpu.sync_copy(ones_v, acc.at[didx.at[j, 0]], add=True)

    plsc.subcore_barrier()
    pltpu.sync_copy(acc.at[pl.ds(sub * RPT, RPT)],
                    out.at[core, pl.ds(sub * RPT, RPT)])


def _sc_deg(dstp):
    mesh = plsc.VectorSubcoreMesh(core_axis_name="c", subcore_axis_name="s")
    cpw = dstp.shape[0] // (NC * NS)
    return pl.kernel(
        _deg_body,
        out_type=jax.ShapeDtypeStruct((NC, N_PAD, 16), jnp.float32),
        mesh=mesh,
        scratch_types=[
            pltpu.VMEM((CH, 16), jnp.float32),
            pltpu.VMEM((CH, 16), jnp.float32),
            pltpu.VMEM((cpw, 1, CH), jnp.int32),
            pltpu.VMEM_SHARED((N_PAD, 16), jnp.float32),
            pltpu.SemaphoreType.DMA,
        ],
    )(dstp)


def _agg_body(xsf, srcp4, dstp, out, sA, dA, sB, dB, rows, acc,
              sg, ss, stA, stB):
    core = lax.axis_index("c")
    sub = lax.axis_index("s")
    nch = dstp.shape[0]
    cpt = nch // NS
    base = sub * cpt
    ngroups = cpt // G
    K = 8

    def zero_rows():
        @pl.loop(0, CH)
        def _z0(i):
            @pl.loop(0, F // 16)
            def _z1(j):
                rows[i, pl.ds(j * 16, 16)] = jnp.zeros((16,), jnp.float32)

    def zero_acc():
        @pl.loop(0, RPT // CH)
        def _z2(k):
            pltpu.sync_copy(rows.at[pl.ds(0, CH)],
                            acc.at[pl.ds(sub * RPT + k * CH, CH)])

    def process(slab_s, slab_d):
        @pl.loop(0, 2)
        def _half(h):
            for k in range(K):
                pltpu.async_copy(xsf.at[slab_s.at[h * K + k, 0]],
                                 rows.at[pl.ds(k * CH, CH)], sg)
            for k in range(K):
                pltpu.make_async_copy(xsf.at[slab_s.at[h * K + k, 0]],
                                      rows.at[pl.ds(k * CH, CH)], sg).wait()
            for k in range(K):
                pltpu.async_copy(rows.at[pl.ds(k * CH, CH)],
                                 acc.at[slab_d.at[h * K + k, 0]], ss,
                                 add=True)
            for k in range(K):
                pltpu.make_async_copy(rows.at[pl.ds(k * CH, CH)],
                                      acc.at[slab_d.at[h * K + k, 0]],
                                      ss).wait()

    zero_rows()
    zero_acc()

    for r in range(2):
        q = core * 2 + r
        pltpu.sync_copy(srcp4.at[q, pl.ds(base, G)], sA)
        pltpu.sync_copy(dstp.at[pl.ds(base, G)], dA)
        plsc.subcore_barrier()

        @pl.loop(0, ngroups, step=2)
        def _grp(gg):
            pltpu.async_copy(srcp4.at[q, pl.ds(base + (gg + 1) * G, G)],
                             sB, stB)
            pltpu.async_copy(dstp.at[pl.ds(base + (gg + 1) * G, G)], dB, stB)
            process(sA, dA)
            pltpu.make_async_copy(srcp4.at[q, pl.ds(base + (gg + 1) * G, G)],
                                  sB, stB).wait()
            pltpu.make_async_copy(dstp.at[pl.ds(base + (gg + 1) * G, G)],
                                  dB, stB).wait()

            @pl.when(gg + 2 < ngroups)
            def _st():
                pltpu.async_copy(srcp4.at[q, pl.ds(base + (gg + 2) * G, G)],
                                 sA, stA)
                pltpu.async_copy(dstp.at[pl.ds(base + (gg + 2) * G, G)],
                                 dA, stA)

            process(sB, dB)

            @pl.when(gg + 2 < ngroups)
            def _sw():
                pltpu.make_async_copy(
                    srcp4.at[q, pl.ds(base + (gg + 2) * G, G)], sA,
                    stA).wait()
                pltpu.make_async_copy(
                    dstp.at[pl.ds(base + (gg + 2) * G, G)], dA, stA).wait()

        plsc.subcore_barrier()

        @pl.loop(0, RPT // CH)
        def _wb(k):
            pltpu.sync_copy(acc.at[pl.ds(sub * RPT + k * CH, CH)],
                            out.at[core, r, pl.ds(sub * RPT + k * CH, CH)])

        plsc.subcore_barrier()

        if r == 0:
            zero_rows()
            zero_acc()


def _sc_agg(xsf, srcp4, dstp):
    mesh = plsc.VectorSubcoreMesh(core_axis_name="c", subcore_axis_name="s")
    return pl.kernel(
        _agg_body,
        out_type=jax.ShapeDtypeStruct((NC, 2, N_PAD, F), jnp.float32),
        mesh=mesh,
        compiler_params=pltpu.CompilerParams(use_tc_tiling_on_sc=False),
        scratch_types=[
            pltpu.VMEM((G, 1, CH), jnp.int32),
            pltpu.VMEM((G, 1, CH), jnp.int32),
            pltpu.VMEM((G, 1, CH), jnp.int32),
            pltpu.VMEM((G, 1, CH), jnp.int32),
            pltpu.VMEM((8 * CH, F), jnp.float32),
            pltpu.VMEM_SHARED((N_PAD, F), jnp.float32),
            pltpu.SemaphoreType.DMA,
            pltpu.SemaphoreType.DMA,
            pltpu.SemaphoreType.DMA,
            pltpu.SemaphoreType.DMA,
        ],
    )(xsf, srcp4, dstp)


def _tc1_body(degp_ref, x1_ref, x2_ref, dinv_ref, xs4_ref):
    deg = degp_ref[0, :, 0:1] + degp_ref[1, :, 0:1] + 1.0
    dinv = lax.rsqrt(deg)
    dinv_ref[...] = dinv
    xsa = x1_ref[...] * dinv
    xsb = x2_ref[...] * dinv
    xs4_ref[0, 0] = xsa[:, :F]
    xs4_ref[0, 1] = xsa[:, F:]
    xs4_ref[1, 0] = xsb[:, :F]
    xs4_ref[1, 1] = xsb[:, F:]


def _tc1(degp, x1p, x2p):
    nb = N_PAD // BLK
    return pl.pallas_call(
        _tc1_body,
        grid=(nb,),
        in_specs=[
            pl.BlockSpec((NC, BLK, 16), lambda i: (0, i, 0)),
            pl.BlockSpec((BLK, D), lambda i: (i, 0)),
            pl.BlockSpec((BLK, D), lambda i: (i, 0)),
        ],
        out_specs=[
            pl.BlockSpec((BLK, 1), lambda i: (i, 0)),
            pl.BlockSpec((NC, 2, BLK, F), lambda i: (0, 0, i, 0)),
        ],
        out_shape=[
            jax.ShapeDtypeStruct((N_PAD, 1), jnp.float32),
            jax.ShapeDtypeStruct((NC, 2, N_PAD, F), jnp.float32),
        ],
    )(degp, x1p, x2p)


def _tc2_body(dinv_ref, agg_ref, xs_ref, w_ref, b_ref, a_ref, out_ref):
    dinv = dinv_ref[...]
    agg = jnp.concatenate([agg_ref[0, 0], agg_ref[0, 1]], axis=1)
    xs = jnp.concatenate([xs_ref[0, 0], xs_ref[0, 1]], axis=1)
    z = dinv * (agg + xs)
    h = jnp.dot(z, w_ref[0], preferred_element_type=jnp.float32) + b_ref[0]
    h = jnp.maximum(h, 0.0) + a_ref[0] * jnp.minimum(h, 0.0)
    xs2 = dinv * h
    out_ref[0, 0] = xs2[:, :F]
    out_ref[0, 1] = xs2[:, F:]


def _tc2(dinv, agg4, xs4, wst, bst, ast):
    nb = N_PAD // BLK
    return pl.pallas_call(
        _tc2_body,
        grid=(NC, nb),
        in_specs=[
            pl.BlockSpec((BLK, 1), lambda c, i: (i, 0)),
            pl.BlockSpec((1, 2, BLK, F), lambda c, i: (c, 0, i, 0)),
            pl.BlockSpec((1, 2, BLK, F), lambda c, i: (c, 0, i, 0)),
            pl.BlockSpec((1, D, D), lambda c, i: (c, 0, 0)),
            pl.BlockSpec((1, 1, D), lambda c, i: (c, 0, 0)),
            pl.BlockSpec((1, 1, D), lambda c, i: (c, 0, 0)),
        ],
        out_specs=pl.BlockSpec((1, 2, BLK, F), lambda c, i: (c, 0, i, 0)),
        out_shape=jax.ShapeDtypeStruct((NC, 2, N_PAD, F), jnp.float32),
    )(dinv, agg4, xs4, wst, bst, ast)


def _tc3_body(dinv_ref, agg_ref, xs_ref, w_ref, b_ref, a_ref, alpha_ref,
              out_ref):
    dinv = dinv_ref[...]

    def tower(c):
        agg = jnp.concatenate([agg_ref[c, 0], agg_ref[c, 1]], axis=1)
        xs = jnp.concatenate([xs_ref[c, 0], xs_ref[c, 1]], axis=1)
        z = dinv * (agg + xs)
        h = jnp.dot(z, w_ref[c], preferred_element_type=jnp.float32) + b_ref[c]
        h = jnp.maximum(h, 0.0) + a_ref[c] * jnp.minimum(h, 0.0)
        nrm = jnp.sqrt(jnp.sum(h * h, axis=1, keepdims=True))
        return h / jnp.maximum(nrm, 1e-12)

    g1 = tower(0)
    g2 = tower(1)
    m = jnp.maximum(alpha_ref[0, 0], alpha_ref[0, 1])
    e0 = jnp.exp(alpha_ref[0, 0] - m)
    e1 = jnp.exp(alpha_ref[0, 1] - m)
    w0 = e0 / (e0 + e1)
    out_ref[...] = g1 * w0 + g2 * (1.0 - w0)


def _tc3(dinv, agg4, xs4, wst, bst, ast, alphap):
    nb = N_PAD // BLK
    return pl.pallas_call(
        _tc3_body,
        grid=(nb,),
        in_specs=[
            pl.BlockSpec((BLK, 1), lambda i: (i, 0)),
            pl.BlockSpec((NC, 2, BLK, F), lambda i: (0, 0, i, 0)),
            pl.BlockSpec((NC, 2, BLK, F), lambda i: (0, 0, i, 0)),
            pl.BlockSpec((NC, D, D), lambda i: (0, 0, 0)),
            pl.BlockSpec((NC, 1, D), lambda i: (0, 0, 0)),
            pl.BlockSpec((NC, 1, D), lambda i: (0, 0, 0)),
            pl.BlockSpec((1, 128), lambda i: (0, 0)),
        ],
        out_specs=pl.BlockSpec((BLK, D), lambda i: (i, 0)),
        out_shape=jax.ShapeDtypeStruct((N_PAD, D), jnp.float32),
    )(dinv, agg4, xs4, wst, bst, ast, alphap)


def kernel(x1, x2, edge_index, W1, b1, W2, b2, W3, b3, W4, b4,
           a1, a2, a3, a4, alpha):
    n = x1.shape[0]

    src = edge_index[0].astype(jnp.int32)
    dst = edge_index[1].astype(jnp.int32)
    e = src.shape[0]
    ep = -(-e // (CH * NS * G)) * (CH * NS * G)
    pad = ep - e
    srcp = jnp.concatenate([src, jnp.full((pad,), n, jnp.int32)])
    dstp = jnp.concatenate([dst, jnp.full((pad,), n, jnp.int32)])
    srcp = srcp.reshape(ep // CH, 1, CH)
    dstp = dstp.reshape(ep // CH, 1, CH)
    srcp4 = (srcp[None] +
             (jnp.arange(4, dtype=jnp.int32) * N_PAD)[:, None, None, None])

    x1p = jnp.pad(x1, ((0, N_PAD - n), (0, 0)))
    x2p = jnp.pad(x2, ((0, N_PAD - n), (0, 0)))

    w12 = jnp.stack([W1, W2])
    b12 = jnp.stack([b1, b2]).reshape(NC, 1, D)
    a13 = jnp.stack([a1, a3]).reshape(NC, 1, D)
    w34 = jnp.stack([W3, W4])
    b34 = jnp.stack([b3, b4]).reshape(NC, 1, D)
    a24 = jnp.stack([a2, a4]).reshape(NC, 1, D)
    alphap = jnp.pad(alpha, (0, 128 - alpha.shape[0])).reshape(1, 128)

    degp = _sc_deg(dstp)
    dinv, xs4 = _tc1(degp, x1p, x2p)

    agg1 = _sc_agg(xs4.reshape(NC * 2 * N_PAD, F), srcp4, dstp)
    xs2 = _tc2(dinv, agg1, xs4, w12, b12, a13)

    agg2 = _sc_agg(xs2.reshape(NC * 2 * N_PAD, F), srcp4, dstp)
    out = _tc3(dinv, agg2, xs2, w34, b34, a24, alphap)

    return out[:n]


# speedup vs baseline: 1.1724x; 1.1724x over previous
import functools

import jax
import jax.numpy as jnp
from jax import lax
from jax.experimental import pallas as pl
from jax.experimental.pallas import tpu as pltpu
from jax.experimental.pallas import tpu_sc as plsc

N_NODES = 10000
D = 128
NC, NS = 2, 16
CH = 128
N_PAD = 10240
RPT = N_PAD // NS
G = 16
BLK = 1024


def _deg_body(dstp, out, ones_v, zb, didx, acc, sdm):
    core = lax.axis_index("c")
    sub = lax.axis_index("s")
    wid = core * NS + sub
    nch = dstp.shape[0]
    cpw = nch // (NC * NS)

    @pl.loop(0, CH)
    def _fill(i):
        ones_v[i, :] = jnp.ones((16,), jnp.float32)
        zb[i, :] = jnp.zeros((16,), jnp.float32)

    @pl.loop(0, RPT // CH)
    def _z(k):
        pltpu.sync_copy(zb, acc.at[pl.ds(sub * RPT + k * CH, CH)])

    pltpu.sync_copy(dstp.at[pl.ds(wid * cpw, cpw)], didx)
    plsc.subcore_barrier()

    @pl.loop(0, cpw)
    def _go(j):
        pltpu.sync_copy(ones_v, acc.at[didx.at[j, 0]], add=True)

    plsc.subcore_barrier()
    pltpu.sync_copy(acc.at[pl.ds(sub * RPT, RPT)],
                    out.at[core, pl.ds(sub * RPT, RPT)])


def _agg_body(xs, srcp2, dstp, out, sidx, didx, rows, acc):
    core = lax.axis_index("c")
    sub = lax.axis_index("s")
    nch = dstp.shape[0]
    cpt = nch // NS

    @pl.loop(0, CH)
    def _z0(i):
        @pl.loop(0, D // 16)
        def _z1(j):
            rows[i, pl.ds(j * 16, 16)] = jnp.zeros((16,), jnp.float32)

    @pl.loop(0, RPT // CH)
    def _z2(k):
        pltpu.sync_copy(rows, acc.at[pl.ds(sub * RPT + k * CH, CH)])

    plsc.subcore_barrier()

    @pl.loop(0, cpt)
    def _go(j):
        q = sub * cpt + j
        pltpu.sync_copy(srcp2.at[core, q], sidx.at[0])
        pltpu.sync_copy(dstp.at[q], didx.at[0])
        pltpu.sync_copy(xs.at[sidx.at[0]], rows)
        pltpu.sync_copy(rows, acc.at[didx.at[0]], add=True)

    plsc.subcore_barrier()

    @pl.loop(0, RPT // CH)
    def _wb(k):
        pltpu.sync_copy(acc.at[pl.ds(sub * RPT + k * CH, CH)],
                        out.at[pl.ds(core * N_PAD + sub * RPT + k * CH, CH)])


def _sc_deg(dstp):
    mesh = plsc.VectorSubcoreMesh(core_axis_name="c", subcore_axis_name="s")
    cpw = dstp.shape[0] // (NC * NS)
    return pl.kernel(
        _deg_body,
        out_type=jax.ShapeDtypeStruct((NC, N_PAD, 16), jnp.float32),
        mesh=mesh,
        scratch_types=[
            pltpu.VMEM((CH, 16), jnp.float32),
            pltpu.VMEM((CH, 16), jnp.float32),
            pltpu.VMEM((cpw, 1, CH), jnp.int32),
            pltpu.VMEM_SHARED((N_PAD, 16), jnp.float32),
            pltpu.SemaphoreType.DMA,
        ],
    )(dstp)


def _sc_agg(xs, srcp2, dstp):
    mesh = plsc.VectorSubcoreMesh(core_axis_name="c", subcore_axis_name="s")
    return pl.kernel(
        _agg_body,
        out_type=jax.ShapeDtypeStruct((NC * N_PAD, D), jnp.float32),
        mesh=mesh,
        scratch_types=[
            pltpu.VMEM((1, CH), jnp.int32),
            pltpu.VMEM((1, CH), jnp.int32),
            pltpu.VMEM((CH, D), jnp.float32),
            pltpu.VMEM_SHARED((N_PAD, D), jnp.float32),
        ],
    )(xs, srcp2, dstp)


def _tc1_body(degp_ref, x1_ref, x2_ref, dinv_ref, xsa_ref, xsb_ref):
    deg = degp_ref[0, :, 0:1] + degp_ref[1, :, 0:1] + 1.0
    dinv = lax.rsqrt(deg)
    dinv_ref[...] = dinv
    xsa_ref[...] = x1_ref[...] * dinv
    xsb_ref[...] = x2_ref[...] * dinv


def _tc1(degp, x1p, x2p):
    nb = N_PAD // BLK
    return pl.pallas_call(
        _tc1_body,
        grid=(nb,),
        in_specs=[
            pl.BlockSpec((NC, BLK, 16), lambda i: (0, i, 0)),
            pl.BlockSpec((BLK, D), lambda i: (i, 0)),
            pl.BlockSpec((BLK, D), lambda i: (i, 0)),
        ],
        out_specs=[
            pl.BlockSpec((BLK, 1), lambda i: (i, 0)),
            pl.BlockSpec((BLK, D), lambda i: (i, 0)),
            pl.BlockSpec((BLK, D), lambda i: (i, 0)),
        ],
        out_shape=[
            jax.ShapeDtypeStruct((N_PAD, 1), jnp.float32),
            jax.ShapeDtypeStruct((N_PAD, D), jnp.float32),
            jax.ShapeDtypeStruct((N_PAD, D), jnp.float32),
        ],
    )(degp, x1p, x2p)


def _tc2_body(dinv_ref, agg_ref, xs_ref, w_ref, b_ref, a_ref, out_ref):
    dinv = dinv_ref[...]
    z = dinv * (agg_ref[0] + xs_ref[0])
    h = jnp.dot(z, w_ref[0], preferred_element_type=jnp.float32) + b_ref[0]
    h = jnp.maximum(h, 0.0) + a_ref[0] * jnp.minimum(h, 0.0)
    out_ref[0] = dinv * h


def _tc2(dinv, agg, xs, wst, bst, ast):
    nb = N_PAD // BLK
    return pl.pallas_call(
        _tc2_body,
        grid=(NC, nb),
        in_specs=[
            pl.BlockSpec((BLK, 1), lambda c, i: (i, 0)),
            pl.BlockSpec((1, BLK, D), lambda c, i: (c, i, 0)),
            pl.BlockSpec((1, BLK, D), lambda c, i: (c, i, 0)),
            pl.BlockSpec((1, D, D), lambda c, i: (c, 0, 0)),
            pl.BlockSpec((1, 1, D), lambda c, i: (c, 0, 0)),
            pl.BlockSpec((1, 1, D), lambda c, i: (c, 0, 0)),
        ],
        out_specs=pl.BlockSpec((1, BLK, D), lambda c, i: (c, i, 0)),
        out_shape=jax.ShapeDtypeStruct((NC, N_PAD, D), jnp.float32),
    )(dinv, agg, xs, wst, bst, ast)


def _tc3_body(dinv_ref, agg_ref, xs_ref, w_ref, b_ref, a_ref, alpha_ref,
              out_ref):
    dinv = dinv_ref[...]

    def tower(c):
        z = dinv * (agg_ref[c] + xs_ref[c])
        h = jnp.dot(z, w_ref[c], preferred_element_type=jnp.float32) + b_ref[c]
        h = jnp.maximum(h, 0.0) + a_ref[c] * jnp.minimum(h, 0.0)
        nrm = jnp.sqrt(jnp.sum(h * h, axis=1, keepdims=True))
        return h / jnp.maximum(nrm, 1e-12)

    g1 = tower(0)
    g2 = tower(1)
    e0 = jnp.exp(alpha_ref[0, 0] - jnp.maximum(alpha_ref[0, 0],
                                               alpha_ref[0, 1]))
    e1 = jnp.exp(alpha_ref[0, 1] - jnp.maximum(alpha_ref[0, 0],
                                               alpha_ref[0, 1]))
    w0 = e0 / (e0 + e1)
    out_ref[...] = g1 * w0 + g2 * (1.0 - w0)


def _tc3(dinv, agg2, xs2, wst, bst, ast, alphap):
    nb = N_PAD // BLK
    return pl.pallas_call(
        _tc3_body,
        grid=(nb,),
        in_specs=[
            pl.BlockSpec((BLK, 1), lambda i: (i, 0)),
            pl.BlockSpec((NC, BLK, D), lambda i: (0, i, 0)),
            pl.BlockSpec((NC, BLK, D), lambda i: (0, i, 0)),
            pl.BlockSpec((NC, D, D), lambda i: (0, 0, 0)),
            pl.BlockSpec((NC, 1, D), lambda i: (0, 0, 0)),
            pl.BlockSpec((NC, 1, D), lambda i: (0, 0, 0)),
            pl.BlockSpec((1, 128), lambda i: (0, 0)),
        ],
        out_specs=pl.BlockSpec((BLK, D), lambda i: (i, 0)),
        out_shape=jax.ShapeDtypeStruct((N_PAD, D), jnp.float32),
    )(dinv, agg2, xs2, wst, bst, ast, alphap)


def kernel(x1, x2, edge_index, W1, b1, W2, b2, W3, b3, W4, b4,
           a1, a2, a3, a4, alpha):
    n = x1.shape[0]

    src = edge_index[0].astype(jnp.int32)
    dst = edge_index[1].astype(jnp.int32)
    e = src.shape[0]
    ep = -(-e // (CH * NC * NS)) * (CH * NC * NS)
    pad = ep - e
    srcp = jnp.concatenate([src, jnp.full((pad,), n, jnp.int32)])
    dstp = jnp.concatenate([dst, jnp.full((pad,), n, jnp.int32)])
    srcp = srcp.reshape(ep // CH, CH)
    dstp = dstp.reshape(ep // CH, CH)
    srcp2 = jnp.stack([srcp, srcp + N_PAD])

    x1p = jnp.pad(x1, ((0, N_PAD - n), (0, 0)))
    x2p = jnp.pad(x2, ((0, N_PAD - n), (0, 0)))

    w12 = jnp.stack([W1, W2])
    b12 = jnp.stack([b1, b2]).reshape(NC, 1, D)
    a13 = jnp.stack([a1, a3]).reshape(NC, 1, D)
    w34 = jnp.stack([W3, W4])
    b34 = jnp.stack([b3, b4]).reshape(NC, 1, D)
    a24 = jnp.stack([a2, a4]).reshape(NC, 1, D)
    alphap = jnp.pad(alpha, (0, 128 - alpha.shape[0])).reshape(1, 128)

    degp = _sc_deg(dstp.reshape(ep // CH, 1, CH))
    dinv, xsa, xsb = _tc1(degp, x1p, x2p)
    xs = jnp.concatenate([xsa, xsb], axis=0)

    agg1 = _sc_agg(xs, srcp2, dstp)
    xs2 = _tc2(dinv, agg1.reshape(NC, N_PAD, D),
               jnp.stack([xsa, xsb]), w12, b12, a13)

    agg2 = _sc_agg(xs2.reshape(NC * N_PAD, D), srcp2, dstp)
    out = _tc3(dinv, agg2.reshape(NC, N_PAD, D), xs2, w34, b34, a24, alphap)

    return out[:n]
